# Initial kernel scaffold; baseline (speedup 1.0000x reference)
import jax
import jax.numpy as jnp
from jax.experimental import pallas as pl


def kernel(input_states, x_embedding, y_embedding, type_embedding):
    raise NotImplementedError("write your pallas kernel here")


# speedup vs baseline: 6.6641x; 6.6641x over previous
import functools

import jax
import jax.numpy as jnp
from jax import lax
from jax.experimental import pallas as pl
from jax.experimental.pallas import tpu as pltpu
from jax.experimental.pallas import tpu_sc as plsc

BATCH, SEQ = 4096, 200
HIDDEN = 128
N = BATCH * SEQ
NC, NS = 2, 16
NW = NC * NS
PER_W = N // NW
T = 128
CHUNKS = PER_W // T
L = 16


def _embed_body(i0_hbm, i1_hbm, i2_hbm, xpad_hbm, ypad_hbm, type_hbm, out_hbm,
                idx_stage, idx_stage1, idx_stage2, idx0, idx1, idx2,
                out_buf, sem):
    wid = lax.axis_index("s") * NC + lax.axis_index("c")
    base = wid * PER_W

    def chunk_body(c, carry):
        pos = base + c * T
        pltpu.sync_copy(i0_hbm.at[pl.ds(pos, T)], idx_stage)
        pltpu.sync_copy(i1_hbm.at[pl.ds(pos, T)], idx_stage1)
        pltpu.sync_copy(i2_hbm.at[pl.ds(pos, T)], idx_stage2)

        def de_body(i, carry2):
            s = pl.ds(i * L, L)
            idx0[s] = idx_stage[s] + 1
            idx1[s] = idx_stage1[s] + 1
            idx2[s] = idx_stage2[s] + 1
            return carry2

        lax.fori_loop(0, T // L, de_body, 0, unroll=True)

        pltpu.async_copy(type_hbm.at[idx2], out_buf, sem).wait()
        pltpu.async_copy(xpad_hbm.at[idx0], out_buf, sem, add=True).wait()
        pltpu.async_copy(ypad_hbm.at[idx1], out_buf, sem, add=True).wait()

        pltpu.sync_copy(out_buf, out_hbm.at[pl.ds(pos, T)])
        return carry

    lax.fori_loop(0, CHUNKS, chunk_body, 0)


@jax.jit
def _embed(i0, i1, i2, xpad, ypad, type_embedding):
    mesh = plsc.VectorSubcoreMesh(core_axis_name="c", subcore_axis_name="s",
                                  num_cores=NC, num_subcores=NS)
    f = pl.kernel(
        _embed_body,
        out_type=jax.ShapeDtypeStruct((N, HIDDEN), jnp.float32),
        mesh=mesh,
        scratch_types=[
            pltpu.VMEM((T,), jnp.int32),
            pltpu.VMEM((T,), jnp.int32),
            pltpu.VMEM((T,), jnp.int32),
            pltpu.VMEM((T,), jnp.int32),
            pltpu.VMEM((T,), jnp.int32),
            pltpu.VMEM((T,), jnp.int32),
            pltpu.VMEM((T, HIDDEN), jnp.float32),
            pltpu.SemaphoreType.DMA,
        ],
    )
    return f(i0, i1, i2, xpad, ypad, type_embedding)


def kernel(input_states, x_embedding, y_embedding, type_embedding):
    inp = input_states.reshape(N, 3).astype(jnp.int32)
    i0 = inp[:, 0]
    i1 = inp[:, 1]
    i2 = inp[:, 2]
    zx = jnp.zeros_like(x_embedding)
    zy = jnp.zeros_like(y_embedding)
    xpad = jnp.concatenate([x_embedding, zx], axis=1)
    ypad = jnp.concatenate([zy, y_embedding], axis=1)
    out = _embed(i0, i1, i2, xpad, ypad, type_embedding)
    return out.reshape(BATCH, SEQ, HIDDEN)


# speedup vs baseline: 11.0809x; 1.6628x over previous
import functools

import jax
import jax.numpy as jnp
from jax import lax
from jax.experimental import pallas as pl
from jax.experimental.pallas import tpu as pltpu
from jax.experimental.pallas import tpu_sc as plsc

BATCH, SEQ = 4096, 200
HIDDEN = 128
N = BATCH * SEQ
NC, NS = 2, 16
NW = NC * NS
PER_W = N // NW
T = 128
CHUNKS = PER_W // T
L = 16
NBUF = 4


def _embed_body(i0_hbm, i1_hbm, i2_hbm, xpad_hbm, ypad_hbm, type_hbm, out_hbm,
                idx0, idx1, idx2, tbuf, ssem, tsem, asem, osem):
    wid = lax.axis_index("s") * NC + lax.axis_index("c")
    base = wid * PER_W

    def stage(c):
        p = c % NBUF
        src = pl.ds(base + c * T, T)
        pltpu.async_copy(i0_hbm.at[src], idx0.at[p], ssem.at[p])
        pltpu.async_copy(i1_hbm.at[src], idx1.at[p], ssem.at[p])
        pltpu.async_copy(i2_hbm.at[src], idx2.at[p], ssem.at[p])

    def wait_stage(p):
        dummy = pl.ds(0, T)
        pltpu.make_async_copy(i0_hbm.at[dummy], idx0.at[p], ssem.at[p]).wait()
        pltpu.make_async_copy(i1_hbm.at[dummy], idx1.at[p], ssem.at[p]).wait()
        pltpu.make_async_copy(i2_hbm.at[dummy], idx2.at[p], ssem.at[p]).wait()

    def bump(p):
        def bbody(i, carry):
            s = pl.ds(i * L, L)
            idx0[p, s] = idx0[p, s] + 1
            idx1[p, s] = idx1[p, s] + 1
            idx2[p, s] = idx2[p, s] + 1
            return carry
        lax.fori_loop(0, T // L, bbody, 0, unroll=True)

    def fire_type(c):
        p = c % NBUF
        pltpu.async_copy(type_hbm.at[idx2.at[p]], tbuf.at[p], tsem.at[p])

    def wait_type(p):
        pltpu.make_async_copy(type_hbm.at[idx2.at[p]], tbuf.at[p],
                              tsem.at[p]).wait()

    def fire_adds(c):
        p = c % NBUF
        pltpu.async_copy(xpad_hbm.at[idx0.at[p]], tbuf.at[p], asem.at[p],
                         add=True)
        pltpu.async_copy(ypad_hbm.at[idx1.at[p]], tbuf.at[p], asem.at[p],
                         add=True)

    def wait_adds(p):
        pltpu.make_async_copy(xpad_hbm.at[idx0.at[p]], tbuf.at[p],
                              asem.at[p]).wait()
        pltpu.make_async_copy(ypad_hbm.at[idx1.at[p]], tbuf.at[p],
                              asem.at[p]).wait()

    def fire_store(c):
        p = c % NBUF
        pltpu.async_copy(tbuf.at[p], out_hbm.at[pl.ds(base + c * T, T)],
                         osem.at[p])

    def wait_store(p):
        pltpu.make_async_copy(tbuf.at[p], out_hbm.at[pl.ds(base, T)],
                              osem.at[p]).wait()

    stage(0)
    stage(1)

    def it(c, carry):
        p = c % NBUF

        @pl.when(jnp.logical_and(c >= 1, c <= CHUNKS))
        def _adds():
            q = (c - 1) % NBUF
            wait_type(q)
            fire_adds(c - 1)

        @pl.when(c >= 2)
        def _store():
            r = (c - 2) % NBUF
            wait_adds(r)
            fire_store(c - 2)

        @pl.when(c < CHUNKS)
        def _front():
            wait_stage(p)
            bump(p)

            @pl.when(c >= NBUF)
            def _reuse():
                wait_store(p)

            fire_type(c)

            @pl.when(c + 2 < CHUNKS)
            def _stage_ahead():
                stage(c + 2)

        return carry

    lax.fori_loop(0, CHUNKS + 2, it, 0)

    for k in range(CHUNKS - NBUF, CHUNKS):
        wait_store(k % NBUF)


@jax.jit
def _embed(i0, i1, i2, xpad, ypad, type_embedding):
    mesh = plsc.VectorSubcoreMesh(core_axis_name="c", subcore_axis_name="s",
                                  num_cores=NC, num_subcores=NS)
    f = pl.kernel(
        _embed_body,
        out_type=jax.ShapeDtypeStruct((N, HIDDEN), jnp.float32),
        mesh=mesh,
        scratch_types=[
            pltpu.VMEM((NBUF, T), jnp.int32),
            pltpu.VMEM((NBUF, T), jnp.int32),
            pltpu.VMEM((NBUF, T), jnp.int32),
            pltpu.VMEM((NBUF, T, HIDDEN), jnp.float32),
            pltpu.SemaphoreType.DMA((NBUF,)),
            pltpu.SemaphoreType.DMA((NBUF,)),
            pltpu.SemaphoreType.DMA((NBUF,)),
            pltpu.SemaphoreType.DMA((NBUF,)),
        ],
    )
    return f(i0, i1, i2, xpad, ypad, type_embedding)


def kernel(input_states, x_embedding, y_embedding, type_embedding):
    inp = input_states.reshape(N, 3).astype(jnp.int32)
    i0 = inp[:, 0]
    i1 = inp[:, 1]
    i2 = inp[:, 2]
    zx = jnp.zeros_like(x_embedding)
    zy = jnp.zeros_like(y_embedding)
    xpad = jnp.concatenate([x_embedding, zx], axis=1)
    ypad = jnp.concatenate([zy, y_embedding], axis=1)
    out = _embed(i0, i1, i2, xpad, ypad, type_embedding)
    return out.reshape(BATCH, SEQ, HIDDEN)


# speedup vs baseline: 12.7674x; 1.1522x over previous
import functools

import jax
import jax.numpy as jnp
from jax import lax
from jax.experimental import pallas as pl
from jax.experimental.pallas import tpu as pltpu
from jax.experimental.pallas import tpu_sc as plsc

BATCH, SEQ = 4096, 200
HIDDEN = 128
HALF = HIDDEN // 2
N = BATCH * SEQ
NC, NS = 2, 16
NW = NC * NS
PER_W = N // NW
T = 128
CHUNKS = PER_W // T
L = 16
NBUF = 4


def _embed_body(i0_hbm, i1_hbm, i2_hbm, x_hbm, y_hbm, tlo_hbm, thi_hbm,
                out_hbm, idx0, idx1, idx2, lobuf, hibuf,
                ssem, tsem, asem, osem):
    wid = lax.axis_index("s") * NC + lax.axis_index("c")
    base = wid * PER_W

    def stage(c):
        p = c % NBUF
        src = pl.ds(base + c * T, T)
        pltpu.async_copy(i0_hbm.at[src], idx0.at[p], ssem.at[p])
        pltpu.async_copy(i1_hbm.at[src], idx1.at[p], ssem.at[p])
        pltpu.async_copy(i2_hbm.at[src], idx2.at[p], ssem.at[p])

    def wait_stage(p):
        dummy = pl.ds(0, T)
        pltpu.make_async_copy(i0_hbm.at[dummy], idx0.at[p], ssem.at[p]).wait()
        pltpu.make_async_copy(i1_hbm.at[dummy], idx1.at[p], ssem.at[p]).wait()
        pltpu.make_async_copy(i2_hbm.at[dummy], idx2.at[p], ssem.at[p]).wait()

    def bump(p):
        def bbody(i, carry):
            s = pl.ds(i * L, L)
            idx0[p, s] = idx0[p, s] + 1
            idx1[p, s] = idx1[p, s] + 1
            idx2[p, s] = idx2[p, s] + 1
            return carry
        lax.fori_loop(0, T // L, bbody, 0, unroll=True)

    def fire_writes(c):
        p = c % NBUF
        pltpu.async_copy(x_hbm.at[idx0.at[p]], lobuf.at[p], tsem.at[p])
        pltpu.async_copy(y_hbm.at[idx1.at[p]], hibuf.at[p], tsem.at[p])

    def wait_writes(p):
        pltpu.make_async_copy(x_hbm.at[idx0.at[p]], lobuf.at[p],
                              tsem.at[p]).wait()
        pltpu.make_async_copy(y_hbm.at[idx1.at[p]], hibuf.at[p],
                              tsem.at[p]).wait()

    def fire_adds(c):
        p = c % NBUF
        pltpu.async_copy(tlo_hbm.at[idx2.at[p]], lobuf.at[p], asem.at[p],
                         add=True)
        pltpu.async_copy(thi_hbm.at[idx2.at[p]], hibuf.at[p], asem.at[p],
                         add=True)

    def wait_adds(p):
        pltpu.make_async_copy(tlo_hbm.at[idx2.at[p]], lobuf.at[p],
                              asem.at[p]).wait()
        pltpu.make_async_copy(thi_hbm.at[idx2.at[p]], hibuf.at[p],
                              asem.at[p]).wait()

    def fire_store(c):
        p = c % NBUF
        rows = pl.ds(base + c * T, T)
        pltpu.async_copy(lobuf.at[p], out_hbm.at[rows, pl.ds(0, HALF)],
                         osem.at[p])
        pltpu.async_copy(hibuf.at[p], out_hbm.at[rows, pl.ds(HALF, HALF)],
                         osem.at[p])

    def wait_store(p):
        rows = pl.ds(base, T)
        pltpu.make_async_copy(lobuf.at[p], out_hbm.at[rows, pl.ds(0, HALF)],
                              osem.at[p]).wait()
        pltpu.make_async_copy(hibuf.at[p], out_hbm.at[rows, pl.ds(HALF, HALF)],
                              osem.at[p]).wait()

    stage(0)
    stage(1)

    def it(c, carry):
        p = c % NBUF

        @pl.when(jnp.logical_and(c >= 1, c <= CHUNKS))
        def _adds():
            q = (c - 1) % NBUF
            wait_writes(q)
            fire_adds(c - 1)

        @pl.when(c >= 2)
        def _store():
            r = (c - 2) % NBUF
            wait_adds(r)
            fire_store(c - 2)

        @pl.when(c < CHUNKS)
        def _front():
            wait_stage(p)
            bump(p)

            @pl.when(c >= NBUF)
            def _reuse():
                wait_store(p)

            fire_writes(c)

            @pl.when(c + 2 < CHUNKS)
            def _stage_ahead():
                stage(c + 2)

        return carry

    lax.fori_loop(0, CHUNKS + 2, it, 0)

    for k in range(CHUNKS - NBUF, CHUNKS):
        wait_store(k % NBUF)


@jax.jit
def _embed(i0, i1, i2, x_embedding, y_embedding, tlo, thi):
    mesh = plsc.VectorSubcoreMesh(core_axis_name="c", subcore_axis_name="s",
                                  num_cores=NC, num_subcores=NS)
    f = pl.kernel(
        _embed_body,
        out_type=jax.ShapeDtypeStruct((N, HIDDEN), jnp.float32),
        mesh=mesh,
        compiler_params=pltpu.CompilerParams(use_tc_tiling_on_sc=False),
        scratch_types=[
            pltpu.VMEM((NBUF, T), jnp.int32),
            pltpu.VMEM((NBUF, T), jnp.int32),
            pltpu.VMEM((NBUF, T), jnp.int32),
            pltpu.VMEM((NBUF, T, HALF), jnp.float32),
            pltpu.VMEM((NBUF, T, HALF), jnp.float32),
            pltpu.SemaphoreType.DMA((NBUF,)),
            pltpu.SemaphoreType.DMA((NBUF,)),
            pltpu.SemaphoreType.DMA((NBUF,)),
            pltpu.SemaphoreType.DMA((NBUF,)),
        ],
    )
    return f(i0, i1, i2, x_embedding, y_embedding, tlo, thi)


def kernel(input_states, x_embedding, y_embedding, type_embedding):
    inp = input_states.reshape(N, 3).astype(jnp.int32)
    i0 = inp[:, 0]
    i1 = inp[:, 1]
    i2 = inp[:, 2]
    tlo = type_embedding[:, :HALF]
    thi = type_embedding[:, HALF:]
    out = _embed(i0, i1, i2, x_embedding, y_embedding, tlo, thi)
    return out.reshape(BATCH, SEQ, HIDDEN)
